# Initial kernel scaffold; baseline (speedup 1.0000x reference)
import jax
import jax.numpy as jnp
from jax.experimental import pallas as pl


def kernel(x, x_complete, mask, edge_src, edge_dst, edge_value, params):
    raise NotImplementedError("write your pallas kernel here")


# speedup vs baseline: 2.8449x; 2.8449x over previous
import functools

import jax
import jax.numpy as jnp
from jax import lax
from jax.experimental import pallas as pl
from jax.experimental.pallas import tpu as pltpu
from jax.experimental.pallas import tpu_sc as plsc

F = 128
NE = 64
MSG = 64
HEADS = 2
TAU = 0.1
N_REAL = 10000
E_REAL = 320000

NP = 10240
CE = 4096
NCHUNK = 80
EP = NCHUNK * CE
NT = 32
ET = EP // NT
KT = ET // 128
ZR = NP // 16

_f32 = jnp.float32


def _leaky(v):
    return jnp.where(v >= 0, v, 0.01 * v)


def _onehot(dst, n_rows):
    return (dst == lax.broadcasted_iota(jnp.int32, (n_rows, F), 1)).astype(_f32)


def _blk0_body(dst_ref, ev_ref, wm1f_ref, wef_ref, rows_ref, m1_ref, e1_ref,
               aggf_ref):
    i = pl.program_id(0)
    dst = dst_ref[...]
    ev = ev_ref[...]
    oh = _onehot(dst, CE)
    rows = rows_ref[...]
    w1, w2, we, c_m2, c_e = (rows[0:1], rows[1:2], rows[2:3], rows[3:4],
                             rows[4:5])
    m1 = jnp.maximum(
        jnp.dot(oh, wm1f_ref[...], preferred_element_type=_f32) + ev * w1, 0.0)
    m1_ref[...] = m1
    e1 = c_e + jnp.dot(oh, wef_ref[...], preferred_element_type=_f32) + ev * we
    e1_ref[...] = _leaky(e1)
    m2 = jnp.maximum(c_m2 + ev * w2, 0.0)
    gidx = i * CE + lax.broadcasted_iota(jnp.int32, (CE, 1), 0)
    m2 = jnp.where(gidx < E_REAL, m2, 0.0)
    part = lax.dot_general(oh, m2, (((0,), (0,)), ((), ())),
                           preferred_element_type=_f32)

    @pl.when(i == 0)
    def _():
        aggf_ref[...] = jnp.zeros_like(aggf_ref)

    aggf_ref[...] += part


def _blk1_body(dst_ref, e1_ref, g_ref, pm1_ref, pe_ref, wm1e_ref, wm2e_ref,
               wee_ref, sel1_ref, sel2_ref, m1_ref, e2_ref, aggf_ref):
    i = pl.program_id(0)
    dst = dst_ref[...]
    e1 = e1_ref[...]
    g = g_ref[...]
    oh = _onehot(dst, CE)
    dot = lambda a, b: jnp.dot(a, b, preferred_element_type=_f32)
    m1 = jnp.maximum(dot(oh, pm1_ref[...]) + dot(e1, wm1e_ref[...]), 0.0)
    m1_ref[...] = m1
    m2 = jnp.maximum(dot(g, sel1_ref[...]) + dot(e1, wm2e_ref[...]), 0.0)
    e2 = dot(g, sel2_ref[...]) + dot(oh, pe_ref[...]) + dot(e1, wee_ref[...])
    e2_ref[...] = _leaky(e2)
    gidx = i * CE + lax.broadcasted_iota(jnp.int32, (CE, 1), 0)
    m2 = jnp.where(gidx < E_REAL, m2, 0.0)
    part = lax.dot_general(oh, m2, (((0,), (0,)), ((), ())),
                           preferred_element_type=_f32)

    @pl.when(i == 0)
    def _():
        aggf_ref[...] = jnp.zeros_like(aggf_ref)

    aggf_ref[...] += part


def _blk2_body(dst_ref, e2_ref, g2_ref, pm1_ref, wm1e_ref, wm2e_ref, m1_ref,
               aggf_ref):
    i = pl.program_id(0)
    dst = dst_ref[...]
    e2 = e2_ref[...]
    oh = _onehot(dst, CE)
    dot = lambda a, b: jnp.dot(a, b, preferred_element_type=_f32)
    m1 = jnp.maximum(dot(oh, pm1_ref[...]) + dot(e2, wm1e_ref[...]), 0.0)
    m1_ref[...] = m1
    m2 = jnp.maximum(g2_ref[...] + dot(e2, wm2e_ref[...]), 0.0)
    gidx = i * CE + lax.broadcasted_iota(jnp.int32, (CE, 1), 0)
    m2 = jnp.where(gidx < E_REAL, m2, 0.0)
    part = lax.dot_general(oh, m2, (((0,), (0,)), ((), ())),
                           preferred_element_type=_f32)

    @pl.when(i == 0)
    def _():
        aggf_ref[...] = jnp.zeros_like(aggf_ref)

    aggf_ref[...] += part


def _edge_chunk_call(body, n_in_chunked, chunk_shapes, const_shapes,
                     out_shapes, out_chunked):
    in_specs = []
    for shp in chunk_shapes:
        in_specs.append(pl.BlockSpec((CE, shp), lambda i: (i, 0)))
    for shp in const_shapes:
        in_specs.append(
            pl.BlockSpec(shp, lambda i, _z=tuple(0 for _ in shp): _z))
    out_specs = []
    for shp, chunked in zip(out_shapes, out_chunked):
        if chunked:
            out_specs.append(pl.BlockSpec((CE, shp[1]), lambda i: (i, 0)))
        else:
            out_specs.append(pl.BlockSpec(shp, lambda i: (0, 0)))
    out_shape = [
        jax.ShapeDtypeStruct((EP, shp[1]) if chunked else shp, _f32)
        for shp, chunked in zip(out_shapes, out_chunked)
    ]
    return pl.pallas_call(
        body,
        grid=(NCHUNK,),
        in_specs=in_specs,
        out_specs=out_specs,
        out_shape=out_shape,
    )


def _sc_mesh():
    return plsc.VectorSubcoreMesh(core_axis_name="c", subcore_axis_name="s")


def _make_sc_gather(width):
    @functools.partial(
        pl.kernel,
        out_type=jax.ShapeDtypeStruct((EP, width), _f32),
        mesh=_sc_mesh(),
        compiler_params=pltpu.CompilerParams(use_tc_tiling_on_sc=False),
        scratch_types=[
            pltpu.VMEM((KT, 128), jnp.int32),
            pltpu.VMEM((128, width), _f32),
            pltpu.SemaphoreType.DMA,
        ],
    )
    def gather(table_hbm, idx_hbm, out_hbm, idx_v, rows_v, sem):
        c = lax.axis_index("c")
        s = lax.axis_index("s")
        t = c * 16 + s
        base = t * KT
        pltpu.sync_copy(idx_hbm.at[pl.ds(base, KT)], idx_v)

        def body(j, carry):
            pltpu.async_copy(table_hbm.at[idx_v.at[j]], rows_v, sem).wait()
            pltpu.sync_copy(rows_v, out_hbm.at[pl.ds((base + j) * 128, 128)])
            return carry

        lax.fori_loop(0, KT, body, 0)

    return gather


def _make_sc_scatter():
    @functools.partial(
        pl.kernel,
        out_type=jax.ShapeDtypeStruct((2 * NP, MSG), _f32),
        mesh=_sc_mesh(),
        compiler_params=pltpu.CompilerParams(use_tc_tiling_on_sc=False),
        scratch_types=[
            pltpu.VMEM((KT, 128), jnp.int32),
            pltpu.VMEM((128, MSG), _f32),
            pltpu.VMEM_SHARED((NP, MSG), _f32),
        ],
    )
    def scatter(idx_hbm, vals_hbm, zeros_hbm, out_hbm, idx_v, val_v, acc):
        c = lax.axis_index("c")
        s = lax.axis_index("s")
        t = c * 16 + s
        pltpu.sync_copy(zeros_hbm.at[pl.ds(s * ZR, ZR)], acc.at[pl.ds(s * ZR, ZR)])
        plsc.subcore_barrier()
        base = t * KT
        pltpu.sync_copy(idx_hbm.at[pl.ds(base, KT)], idx_v)

        def body(j, carry):
            pltpu.sync_copy(vals_hbm.at[pl.ds((base + j) * 128, 128)], val_v)
            pltpu.sync_copy(val_v, acc.at[idx_v.at[j]], add=True)
            return carry

        lax.fori_loop(0, KT, body, 0)
        plsc.subcore_barrier()
        pltpu.sync_copy(acc.at[pl.ds(s * ZR, ZR)],
                        out_hbm.at[pl.ds(c * NP + s * ZR, ZR)])

    return scatter


def _upd0_body(aggn_ref, aggf_ref, cn_ref, wnb_ref, wftop_ref, wfb_ref,
               wnodecat_ref, wm1t_ref, wet_ref, node1_ref, qq1_ref, feat1_ref,
               pm1_ref, pe_ref):
    dot = lambda a, b: jnp.dot(a, b, preferred_element_type=_f32)
    aggn = aggn_ref[0:NP, :] + aggn_ref[NP:2 * NP, :]
    node1 = _leaky(cn_ref[...] + dot(aggn, wnb_ref[...]))
    node1_ref[...] = node1
    qq1_ref[...] = dot(node1, wnodecat_ref[...])
    feat1 = _leaky(wftop_ref[...] + dot(aggf_ref[...], wfb_ref[...]))
    feat1_ref[...] = feat1
    pm1_ref[...] = dot(feat1, wm1t_ref[...])
    pe_ref[...] = dot(feat1, wet_ref[...])


def _upd1_body(aggn_ref, aggf_ref, node1_ref, feat1_ref, wnt_ref, wnb_ref,
               wft_ref, wfb_ref, wm2t_ref, wm1t_ref, node2_ref, q2_ref,
               feat2_ref, pm1_ref):
    dot = lambda a, b: jnp.dot(a, b, preferred_element_type=_f32)
    aggn = aggn_ref[0:NP, :] + aggn_ref[NP:2 * NP, :]
    node2 = _leaky(dot(node1_ref[...], wnt_ref[...]) + dot(aggn, wnb_ref[...]))
    node2_ref[...] = node2
    q2_ref[...] = dot(node2, wm2t_ref[...])
    feat2 = _leaky(dot(feat1_ref[...], wft_ref[...])
                   + dot(aggf_ref[...], wfb_ref[...]))
    feat2_ref[...] = feat2
    pm1_ref[...] = dot(feat2, wm1t_ref[...])


def _upd2_body(aggn_ref, aggf_ref, node2_ref, feat2_ref, wnt_ref, wnb_ref,
               wft_ref, wfb_ref, logits_ref, wq_ref, wk_ref, node3n_ref,
               cmat_ref, probs_ref, kl_ref):
    dot = lambda a, b: jnp.dot(a, b, preferred_element_type=_f32)
    aggn = aggn_ref[0:NP, :] + aggn_ref[NP:2 * NP, :]
    node3 = _leaky(dot(node2_ref[...], wnt_ref[...]) + dot(aggn, wnb_ref[...]))
    nrm = jnp.sqrt(jnp.sum(node3 * node3, axis=1, keepdims=True))
    node3n = node3 / (nrm + 1e-12)
    node3n_ref[...] = node3n
    feat3 = _leaky(dot(feat2_ref[...], wft_ref[...])
                   + dot(aggf_ref[...], wfb_ref[...]))
    fnrm = jnp.sqrt(jnp.sum(feat3 * feat3, axis=1, keepdims=True))
    feat3n = feat3 / (fnrm + 1e-12)
    probs = jax.nn.sigmoid(logits_ref[...] / TAU)
    probs_ref[...] = probs
    kl = probs * jnp.log(probs / 0.5 + 1e-12) \
        + (1.0 - probs) * jnp.log((1.0 - probs) / 0.5 + 1e-12)
    kl_ref[...] = jnp.sum(kl, keepdims=True).reshape(1, 1) / (F * F)
    ctx = dot(probs, feat3n)
    b = dot(ctx, wk_ref[...])
    scale = 1.0 / (HEADS * jnp.sqrt(jnp.float32(NE // HEADS)))
    cmat_ref[...] = lax.dot_general(wq_ref[...], b, (((1,), (1,)), ((), ())),
                                    preferred_element_type=_f32) * scale


NROW = 2000


def _head_body(node3n_ref, x_ref, xc_ref, mask_ref, cmat_ref, rb_ref, nw_ref,
               nb_ref, dhat_ref, adj_ref, y_ref):
    dot = lambda a, b: jnp.dot(a, b, preferred_element_type=_f32)
    d = dot(node3n_ref[...], cmat_ref[...]) + rb_ref[...]
    dhat_ref[...] = d
    adj_ref[...] = jnp.where(mask_ref[...] == 1, x_ref[...], d)
    y_ref[...] = dot(xc_ref[...], nw_ref[...]) + nb_ref[...]


def kernel(x, x_complete, mask, edge_src, edge_dst, edge_value, params):
    p = params

    pad = EP - E_REAL
    src_pad = jnp.concatenate(
        [edge_src, jnp.full((pad,), N_REAL, jnp.int32)]).astype(jnp.int32)
    dst_pad = jnp.concatenate([edge_dst, jnp.zeros((pad,), jnp.int32)])
    ev_pad = jnp.concatenate([edge_value, jnp.zeros((pad,), _f32)])
    src2d = src_pad.reshape(EP // 128, 128)
    dst_col = dst_pad.reshape(EP, 1)
    ev_col = ev_pad.reshape(EP, 1)
    zeros_np = jnp.zeros((NP, MSG), _f32)

    wm1_0, wm2_0, wn_0, wf_0, we_0 = (p["b0_Wm1"], p["b0_Wm2"], p["b0_Wn"],
                                      p["b0_Wf"], p["b0_We"])
    rows0 = jnp.zeros((8, MSG), _f32)
    rows0 = rows0.at[0].set(wm1_0[F])
    rows0 = rows0.at[1].set(wm2_0[F].astype(_f32))
    rows0 = rows0.at[2].set(we_0[2 * F])
    rows0 = rows0.at[3].set(wm2_0[:F].sum(0))
    rows0 = rows0.at[4].set(we_0[:F].sum(0))
    cn0 = wn_0[:F].sum(0).reshape(1, NE)

    wm1_1, wm2_1, wn_1, wf_1, we_1 = (p["b1_Wm1"], p["b1_Wm2"], p["b1_Wn"],
                                      p["b1_Wf"], p["b1_We"])
    wnodecat1 = jnp.concatenate([wm2_1[:NE], we_1[:NE]], axis=1)
    eye = jnp.eye(NE, dtype=_f32)
    zb = jnp.zeros((NE, NE), _f32)
    sel1 = jnp.concatenate([eye, zb], axis=0)
    sel2 = jnp.concatenate([zb, eye], axis=0)

    wm1_2, wm2_2, wn_2, wf_2 = (p["b2_Wm1"], p["b2_Wm2"], p["b2_Wn"],
                                p["b2_Wf"])

    sc_gather128 = _make_sc_gather(128)
    sc_gather64 = _make_sc_gather(64)
    sc_scatter = _make_sc_scatter()

    blk0 = _edge_chunk_call(
        _blk0_body, 2, [1, 1], [(F, MSG), (F, MSG), (8, MSG)],
        [(EP, MSG), (EP, MSG), (F, MSG)], [True, True, False])
    m1_0, e1, aggf0 = blk0(dst_col, ev_col, wm1_0[:F], we_0[F:2 * F], rows0)

    aggn0 = sc_scatter(src2d, m1_0, zeros_np)

    upd0 = pl.pallas_call(
        _upd0_body,
        out_shape=[
            jax.ShapeDtypeStruct((NP, NE), _f32),
            jax.ShapeDtypeStruct((NP, 2 * NE), _f32),
            jax.ShapeDtypeStruct((F, NE), _f32),
            jax.ShapeDtypeStruct((F, MSG), _f32),
            jax.ShapeDtypeStruct((F, NE), _f32),
        ],
    )
    node1, qq1, feat1, pm1_1, pe_1 = upd0(
        aggn0, aggf0, cn0, wn_0[F:], wf_0[:F], wf_0[F:], wnodecat1,
        wm1_1[:NE], we_1[NE:2 * NE])

    g1 = sc_gather128(qq1, src2d)

    blk1 = _edge_chunk_call(
        _blk1_body, 3, [1, NE, 2 * NE],
        [(F, MSG), (F, NE), (NE, MSG), (NE, MSG), (NE, NE), (2 * NE, NE),
         (2 * NE, NE)],
        [(EP, MSG), (EP, NE), (F, MSG)], [True, True, False])
    m1_1, e2, aggf1 = blk1(dst_col, e1, g1, pm1_1, pe_1, wm1_1[NE:],
                           wm2_1[NE:], we_1[2 * NE:], sel1, sel2)

    aggn1 = sc_scatter(src2d, m1_1, zeros_np)

    upd1 = pl.pallas_call(
        _upd1_body,
        out_shape=[
            jax.ShapeDtypeStruct((NP, NE), _f32),
            jax.ShapeDtypeStruct((NP, NE), _f32),
            jax.ShapeDtypeStruct((F, NE), _f32),
            jax.ShapeDtypeStruct((F, MSG), _f32),
        ],
    )
    node2, q2, feat2, pm1_2 = upd1(
        aggn1, aggf1, node1, feat1, wn_1[:NE], wn_1[NE:], wf_1[:NE],
        wf_1[NE:], wm2_2[:NE], wm1_2[:NE])

    g2 = sc_gather64(q2, src2d)

    blk2 = _edge_chunk_call(
        _blk2_body, 3, [1, NE, NE],
        [(F, MSG), (NE, MSG), (NE, MSG)],
        [(EP, MSG), (F, MSG)], [True, False])
    m1_2, aggf2 = blk2(dst_col, e2, g2, pm1_2, wm1_2[NE:], wm2_2[NE:])

    aggn2 = sc_scatter(src2d, m1_2, zeros_np)

    upd2 = pl.pallas_call(
        _upd2_body,
        out_shape=[
            jax.ShapeDtypeStruct((NP, NE), _f32),
            jax.ShapeDtypeStruct((NE, F), _f32),
            jax.ShapeDtypeStruct((F, F), _f32),
            jax.ShapeDtypeStruct((1, 1), _f32),
        ],
    )
    node3n, cmat, probs, kl = upd2(
        aggn2, aggf2, node2, feat2, wn_2[:NE], wn_2[NE:], wf_2[:NE],
        wf_2[NE:], p["gll_logits"], p["reph_Wq"], p["reph_Wk"])

    head = pl.pallas_call(
        _head_body,
        grid=(N_REAL // NROW,),
        in_specs=[
            pl.BlockSpec((NROW, NE), lambda i: (i, 0)),
            pl.BlockSpec((NROW, F), lambda i: (i, 0)),
            pl.BlockSpec((NROW, F), lambda i: (i, 0)),
            pl.BlockSpec((NROW, F), lambda i: (i, 0)),
            pl.BlockSpec((NE, F), lambda i: (0, 0)),
            pl.BlockSpec((1, F), lambda i: (0, 0)),
            pl.BlockSpec((F, 10), lambda i: (0, 0)),
            pl.BlockSpec((1, 10), lambda i: (0, 0)),
        ],
        out_specs=[
            pl.BlockSpec((NROW, F), lambda i: (i, 0)),
            pl.BlockSpec((NROW, F), lambda i: (i, 0)),
            pl.BlockSpec((NROW, 10), lambda i: (i, 0)),
        ],
        out_shape=[
            jax.ShapeDtypeStruct((N_REAL, F), _f32),
            jax.ShapeDtypeStruct((N_REAL, F), _f32),
            jax.ShapeDtypeStruct((N_REAL, 10), _f32),
        ],
    )
    d_hat, d_hat_adj, y_hat = head(
        node3n[:N_REAL], x, x_complete, mask.astype(jnp.int32),
        cmat, p["reph_b"].reshape(1, F), p["nph_W"],
        p["nph_b"].reshape(1, 10))

    return d_hat, d_hat_adj, y_hat, kl.reshape(()), probs


# speedup vs baseline: 3.1645x; 1.1124x over previous
import functools

import jax
import jax.numpy as jnp
from jax import lax
from jax.experimental import pallas as pl
from jax.experimental.pallas import tpu as pltpu
from jax.experimental.pallas import tpu_sc as plsc

F = 128
NE = 64
MSG = 64
HEADS = 2
TAU = 0.1
N_REAL = 10000
E_REAL = 320000

NP = 10240
CE = 4096
NCHUNK = 80
EP = NCHUNK * CE
NT = 32
ET = EP // NT
KT = ET // 128
ZR = NP // 16

_f32 = jnp.float32


def _leaky(v):
    return jnp.where(v >= 0, v, 0.01 * v)


def _onehot(dst, n_rows):
    return (dst == lax.broadcasted_iota(jnp.int32, (n_rows, F), 1)).astype(_f32)


def _blk0_body(dst_ref, ev_ref, wm1f_ref, wef_ref, rows_ref, m1_ref, e1_ref,
               aggf_ref):
    i = pl.program_id(0)
    dst = dst_ref[...]
    ev = ev_ref[...]
    oh = _onehot(dst, CE)
    rows = rows_ref[...]
    w1, w2, we, c_m2, c_e = (rows[0:1], rows[1:2], rows[2:3], rows[3:4],
                             rows[4:5])
    m1 = jnp.maximum(
        jnp.dot(oh, wm1f_ref[...], preferred_element_type=_f32) + ev * w1, 0.0)
    m1_ref[...] = m1
    e1 = c_e + jnp.dot(oh, wef_ref[...], preferred_element_type=_f32) + ev * we
    e1_ref[...] = _leaky(e1)
    m2 = jnp.maximum(c_m2 + ev * w2, 0.0)
    gidx = i * CE + lax.broadcasted_iota(jnp.int32, (CE, 1), 0)
    m2 = jnp.where(gidx < E_REAL, m2, 0.0)
    part = lax.dot_general(oh, m2, (((0,), (0,)), ((), ())),
                           preferred_element_type=_f32)

    @pl.when(i == 0)
    def _():
        aggf_ref[...] = jnp.zeros_like(aggf_ref)

    aggf_ref[...] += part


def _blk1_body(dst_ref, e1_ref, g_ref, pm1_ref, pe_ref, wm1e_ref, wm2e_ref,
               wee_ref, sel1_ref, sel2_ref, m1_ref, e2_ref, aggf_ref):
    i = pl.program_id(0)
    dst = dst_ref[...]
    e1 = e1_ref[...]
    g = g_ref[...]
    oh = _onehot(dst, CE)
    dot = lambda a, b: jnp.dot(a, b, preferred_element_type=_f32)
    m1 = jnp.maximum(dot(oh, pm1_ref[...]) + dot(e1, wm1e_ref[...]), 0.0)
    m1_ref[...] = m1
    m2 = jnp.maximum(dot(g, sel1_ref[...]) + dot(e1, wm2e_ref[...]), 0.0)
    e2 = dot(g, sel2_ref[...]) + dot(oh, pe_ref[...]) + dot(e1, wee_ref[...])
    e2_ref[...] = _leaky(e2)
    gidx = i * CE + lax.broadcasted_iota(jnp.int32, (CE, 1), 0)
    m2 = jnp.where(gidx < E_REAL, m2, 0.0)
    part = lax.dot_general(oh, m2, (((0,), (0,)), ((), ())),
                           preferred_element_type=_f32)

    @pl.when(i == 0)
    def _():
        aggf_ref[...] = jnp.zeros_like(aggf_ref)

    aggf_ref[...] += part


def _blk2_body(dst_ref, e2_ref, g2_ref, pm1_ref, wm1e_ref, wm2e_ref, m1_ref,
               aggf_ref):
    i = pl.program_id(0)
    dst = dst_ref[...]
    e2 = e2_ref[...]
    oh = _onehot(dst, CE)
    dot = lambda a, b: jnp.dot(a, b, preferred_element_type=_f32)
    m1 = jnp.maximum(dot(oh, pm1_ref[...]) + dot(e2, wm1e_ref[...]), 0.0)
    m1_ref[...] = m1
    m2 = jnp.maximum(g2_ref[...] + dot(e2, wm2e_ref[...]), 0.0)
    gidx = i * CE + lax.broadcasted_iota(jnp.int32, (CE, 1), 0)
    m2 = jnp.where(gidx < E_REAL, m2, 0.0)
    part = lax.dot_general(oh, m2, (((0,), (0,)), ((), ())),
                           preferred_element_type=_f32)

    @pl.when(i == 0)
    def _():
        aggf_ref[...] = jnp.zeros_like(aggf_ref)

    aggf_ref[...] += part


def _edge_chunk_call(body, n_in_chunked, chunk_shapes, const_shapes,
                     out_shapes, out_chunked):
    in_specs = []
    for shp in chunk_shapes:
        in_specs.append(pl.BlockSpec((CE, shp), lambda i: (i, 0)))
    for shp in const_shapes:
        in_specs.append(
            pl.BlockSpec(shp, lambda i, _z=tuple(0 for _ in shp): _z))
    out_specs = []
    for shp, chunked in zip(out_shapes, out_chunked):
        if chunked:
            out_specs.append(pl.BlockSpec((CE, shp[1]), lambda i: (i, 0)))
        else:
            out_specs.append(pl.BlockSpec(shp, lambda i: (0, 0)))
    out_shape = [
        jax.ShapeDtypeStruct((EP, shp[1]) if chunked else shp, _f32)
        for shp, chunked in zip(out_shapes, out_chunked)
    ]
    return pl.pallas_call(
        body,
        grid=(NCHUNK,),
        in_specs=in_specs,
        out_specs=out_specs,
        out_shape=out_shape,
    )


def _sc_mesh():
    return plsc.VectorSubcoreMesh(core_axis_name="c", subcore_axis_name="s")


def _make_sc_gather(width):
    @functools.partial(
        pl.kernel,
        out_type=jax.ShapeDtypeStruct((EP, width), _f32),
        mesh=_sc_mesh(),
        compiler_params=pltpu.CompilerParams(use_tc_tiling_on_sc=False),
        scratch_types=[
            pltpu.VMEM((KT, 128), jnp.int32),
            pltpu.VMEM((128, width), _f32),
            pltpu.VMEM((128, width), _f32),
            pltpu.SemaphoreType.DMA,
            pltpu.SemaphoreType.DMA,
        ],
    )
    def gather(table_hbm, idx_hbm, out_hbm, idx_v, b0, b1, s0, s1):
        c = lax.axis_index("c")
        s = lax.axis_index("s")
        t = c * 16 + s
        base = t * KT
        pltpu.sync_copy(idx_hbm.at[pl.ds(base, KT)], idx_v)
        pltpu.async_copy(table_hbm.at[idx_v.at[0]], b0, s0)

        def body(h, carry):
            j0 = 2 * h
            d1 = pltpu.async_copy(table_hbm.at[idx_v.at[j0 + 1]], b1, s1)
            pltpu.make_async_copy(table_hbm.at[idx_v.at[j0]], b0, s0).wait()
            pltpu.sync_copy(b0, out_hbm.at[pl.ds((base + j0) * 128, 128)])

            @pl.when(h + 1 < KT // 2)
            def _():
                pltpu.async_copy(table_hbm.at[idx_v.at[j0 + 2]], b0, s0)

            d1.wait()
            pltpu.sync_copy(b1, out_hbm.at[pl.ds((base + j0 + 1) * 128, 128)])
            return carry

        lax.fori_loop(0, KT // 2, body, 0)

    return gather


def _make_sc_scatter():
    @functools.partial(
        pl.kernel,
        out_type=jax.ShapeDtypeStruct((2 * NP, MSG), _f32),
        mesh=_sc_mesh(),
        compiler_params=pltpu.CompilerParams(use_tc_tiling_on_sc=False),
        scratch_types=[
            pltpu.VMEM((KT, 128), jnp.int32),
            pltpu.VMEM((128, MSG), _f32),
            pltpu.VMEM((128, MSG), _f32),
            pltpu.VMEM_SHARED((NP, MSG), _f32),
            pltpu.SemaphoreType.DMA,
            pltpu.SemaphoreType.DMA,
        ],
    )
    def scatter(idx_hbm, vals_hbm, zeros_hbm, out_hbm, idx_v, v0, v1, acc,
                s0, s1):
        c = lax.axis_index("c")
        s = lax.axis_index("s")
        t = c * 16 + s
        pltpu.sync_copy(zeros_hbm.at[pl.ds(s * ZR, ZR)], acc.at[pl.ds(s * ZR, ZR)])
        plsc.subcore_barrier()
        base = t * KT
        pltpu.sync_copy(idx_hbm.at[pl.ds(base, KT)], idx_v)
        pltpu.async_copy(vals_hbm.at[pl.ds(base * 128, 128)], v0, s0)

        def body(h, carry):
            j0 = 2 * h
            d1 = pltpu.async_copy(vals_hbm.at[pl.ds((base + j0 + 1) * 128, 128)],
                                  v1, s1)
            pltpu.make_async_copy(vals_hbm.at[pl.ds(base * 128, 128)], v0,
                                  s0).wait()
            pltpu.sync_copy(v0, acc.at[idx_v.at[j0]], add=True)

            @pl.when(h + 1 < KT // 2)
            def _():
                pltpu.async_copy(vals_hbm.at[pl.ds((base + j0 + 2) * 128, 128)],
                                 v0, s0)

            d1.wait()
            pltpu.sync_copy(v1, acc.at[idx_v.at[j0 + 1]], add=True)
            return carry

        lax.fori_loop(0, KT // 2, body, 0)
        plsc.subcore_barrier()
        pltpu.sync_copy(acc.at[pl.ds(s * ZR, ZR)],
                        out_hbm.at[pl.ds(c * NP + s * ZR, ZR)])

    return scatter


def _upd0_body(aggn_ref, aggf_ref, cn_ref, wnb_ref, wftop_ref, wfb_ref,
               wnodecat_ref, wm1t_ref, wet_ref, node1_ref, qq1_ref, feat1_ref,
               pm1_ref, pe_ref):
    dot = lambda a, b: jnp.dot(a, b, preferred_element_type=_f32)
    aggn = aggn_ref[0:NP, :] + aggn_ref[NP:2 * NP, :]
    node1 = _leaky(cn_ref[...] + dot(aggn, wnb_ref[...]))
    node1_ref[...] = node1
    qq1_ref[...] = dot(node1, wnodecat_ref[...])
    feat1 = _leaky(wftop_ref[...] + dot(aggf_ref[...], wfb_ref[...]))
    feat1_ref[...] = feat1
    pm1_ref[...] = dot(feat1, wm1t_ref[...])
    pe_ref[...] = dot(feat1, wet_ref[...])


def _upd1_body(aggn_ref, aggf_ref, node1_ref, feat1_ref, wnt_ref, wnb_ref,
               wft_ref, wfb_ref, wm2t_ref, wm1t_ref, node2_ref, q2_ref,
               feat2_ref, pm1_ref):
    dot = lambda a, b: jnp.dot(a, b, preferred_element_type=_f32)
    aggn = aggn_ref[0:NP, :] + aggn_ref[NP:2 * NP, :]
    node2 = _leaky(dot(node1_ref[...], wnt_ref[...]) + dot(aggn, wnb_ref[...]))
    node2_ref[...] = node2
    q2_ref[...] = dot(node2, wm2t_ref[...])
    feat2 = _leaky(dot(feat1_ref[...], wft_ref[...])
                   + dot(aggf_ref[...], wfb_ref[...]))
    feat2_ref[...] = feat2
    pm1_ref[...] = dot(feat2, wm1t_ref[...])


def _upd2_body(aggn_ref, aggf_ref, node2_ref, feat2_ref, wnt_ref, wnb_ref,
               wft_ref, wfb_ref, logits_ref, wq_ref, wk_ref, node3n_ref,
               cmat_ref, probs_ref, kl_ref):
    dot = lambda a, b: jnp.dot(a, b, preferred_element_type=_f32)
    aggn = aggn_ref[0:NP, :] + aggn_ref[NP:2 * NP, :]
    node3 = _leaky(dot(node2_ref[...], wnt_ref[...]) + dot(aggn, wnb_ref[...]))
    nrm = jnp.sqrt(jnp.sum(node3 * node3, axis=1, keepdims=True))
    node3n = node3 / (nrm + 1e-12)
    node3n_ref[...] = node3n
    feat3 = _leaky(dot(feat2_ref[...], wft_ref[...])
                   + dot(aggf_ref[...], wfb_ref[...]))
    fnrm = jnp.sqrt(jnp.sum(feat3 * feat3, axis=1, keepdims=True))
    feat3n = feat3 / (fnrm + 1e-12)
    probs = jax.nn.sigmoid(logits_ref[...] / TAU)
    probs_ref[...] = probs
    kl = probs * jnp.log(probs / 0.5 + 1e-12) \
        + (1.0 - probs) * jnp.log((1.0 - probs) / 0.5 + 1e-12)
    kl_ref[...] = jnp.sum(kl, keepdims=True).reshape(1, 1) / (F * F)
    ctx = dot(probs, feat3n)
    b = dot(ctx, wk_ref[...])
    scale = 1.0 / (HEADS * jnp.sqrt(jnp.float32(NE // HEADS)))
    cmat_ref[...] = lax.dot_general(wq_ref[...], b, (((1,), (1,)), ((), ())),
                                    preferred_element_type=_f32) * scale


NROW = 2000


def _head_body(node3n_ref, x_ref, xc_ref, mask_ref, cmat_ref, rb_ref, nw_ref,
               nb_ref, dhat_ref, adj_ref, y_ref):
    dot = lambda a, b: jnp.dot(a, b, preferred_element_type=_f32)
    d = dot(node3n_ref[...], cmat_ref[...]) + rb_ref[...]
    dhat_ref[...] = d
    adj_ref[...] = jnp.where(mask_ref[...] == 1, x_ref[...], d)
    y_ref[...] = dot(xc_ref[...], nw_ref[...]) + nb_ref[...]


def kernel(x, x_complete, mask, edge_src, edge_dst, edge_value, params):
    p = params

    pad = EP - E_REAL
    src_pad = jnp.concatenate(
        [edge_src, jnp.full((pad,), N_REAL, jnp.int32)]).astype(jnp.int32)
    dst_pad = jnp.concatenate([edge_dst, jnp.zeros((pad,), jnp.int32)])
    ev_pad = jnp.concatenate([edge_value, jnp.zeros((pad,), _f32)])
    src2d = src_pad.reshape(EP // 128, 128)
    dst_col = dst_pad.reshape(EP, 1)
    ev_col = ev_pad.reshape(EP, 1)
    zeros_np = jnp.zeros((NP, MSG), _f32)

    wm1_0, wm2_0, wn_0, wf_0, we_0 = (p["b0_Wm1"], p["b0_Wm2"], p["b0_Wn"],
                                      p["b0_Wf"], p["b0_We"])
    rows0 = jnp.zeros((8, MSG), _f32)
    rows0 = rows0.at[0].set(wm1_0[F])
    rows0 = rows0.at[1].set(wm2_0[F].astype(_f32))
    rows0 = rows0.at[2].set(we_0[2 * F])
    rows0 = rows0.at[3].set(wm2_0[:F].sum(0))
    rows0 = rows0.at[4].set(we_0[:F].sum(0))
    cn0 = wn_0[:F].sum(0).reshape(1, NE)

    wm1_1, wm2_1, wn_1, wf_1, we_1 = (p["b1_Wm1"], p["b1_Wm2"], p["b1_Wn"],
                                      p["b1_Wf"], p["b1_We"])
    wnodecat1 = jnp.concatenate([wm2_1[:NE], we_1[:NE]], axis=1)
    eye = jnp.eye(NE, dtype=_f32)
    zb = jnp.zeros((NE, NE), _f32)
    sel1 = jnp.concatenate([eye, zb], axis=0)
    sel2 = jnp.concatenate([zb, eye], axis=0)

    wm1_2, wm2_2, wn_2, wf_2 = (p["b2_Wm1"], p["b2_Wm2"], p["b2_Wn"],
                                p["b2_Wf"])

    sc_gather128 = _make_sc_gather(128)
    sc_gather64 = _make_sc_gather(64)
    sc_scatter = _make_sc_scatter()

    blk0 = _edge_chunk_call(
        _blk0_body, 2, [1, 1], [(F, MSG), (F, MSG), (8, MSG)],
        [(EP, MSG), (EP, MSG), (F, MSG)], [True, True, False])
    m1_0, e1, aggf0 = blk0(dst_col, ev_col, wm1_0[:F], we_0[F:2 * F], rows0)

    aggn0 = sc_scatter(src2d, m1_0, zeros_np)

    upd0 = pl.pallas_call(
        _upd0_body,
        out_shape=[
            jax.ShapeDtypeStruct((NP, NE), _f32),
            jax.ShapeDtypeStruct((NP, 2 * NE), _f32),
            jax.ShapeDtypeStruct((F, NE), _f32),
            jax.ShapeDtypeStruct((F, MSG), _f32),
            jax.ShapeDtypeStruct((F, NE), _f32),
        ],
    )
    node1, qq1, feat1, pm1_1, pe_1 = upd0(
        aggn0, aggf0, cn0, wn_0[F:], wf_0[:F], wf_0[F:], wnodecat1,
        wm1_1[:NE], we_1[NE:2 * NE])

    g1 = sc_gather128(qq1, src2d)

    blk1 = _edge_chunk_call(
        _blk1_body, 3, [1, NE, 2 * NE],
        [(F, MSG), (F, NE), (NE, MSG), (NE, MSG), (NE, NE), (2 * NE, NE),
         (2 * NE, NE)],
        [(EP, MSG), (EP, NE), (F, MSG)], [True, True, False])
    m1_1, e2, aggf1 = blk1(dst_col, e1, g1, pm1_1, pe_1, wm1_1[NE:],
                           wm2_1[NE:], we_1[2 * NE:], sel1, sel2)

    aggn1 = sc_scatter(src2d, m1_1, zeros_np)

    upd1 = pl.pallas_call(
        _upd1_body,
        out_shape=[
            jax.ShapeDtypeStruct((NP, NE), _f32),
            jax.ShapeDtypeStruct((NP, NE), _f32),
            jax.ShapeDtypeStruct((F, NE), _f32),
            jax.ShapeDtypeStruct((F, MSG), _f32),
        ],
    )
    node2, q2, feat2, pm1_2 = upd1(
        aggn1, aggf1, node1, feat1, wn_1[:NE], wn_1[NE:], wf_1[:NE],
        wf_1[NE:], wm2_2[:NE], wm1_2[:NE])

    g2 = sc_gather64(q2, src2d)

    blk2 = _edge_chunk_call(
        _blk2_body, 3, [1, NE, NE],
        [(F, MSG), (NE, MSG), (NE, MSG)],
        [(EP, MSG), (F, MSG)], [True, False])
    m1_2, aggf2 = blk2(dst_col, e2, g2, pm1_2, wm1_2[NE:], wm2_2[NE:])

    aggn2 = sc_scatter(src2d, m1_2, zeros_np)

    upd2 = pl.pallas_call(
        _upd2_body,
        out_shape=[
            jax.ShapeDtypeStruct((NP, NE), _f32),
            jax.ShapeDtypeStruct((NE, F), _f32),
            jax.ShapeDtypeStruct((F, F), _f32),
            jax.ShapeDtypeStruct((1, 1), _f32),
        ],
    )
    node3n, cmat, probs, kl = upd2(
        aggn2, aggf2, node2, feat2, wn_2[:NE], wn_2[NE:], wf_2[:NE],
        wf_2[NE:], p["gll_logits"], p["reph_Wq"], p["reph_Wk"])

    head = pl.pallas_call(
        _head_body,
        grid=(N_REAL // NROW,),
        in_specs=[
            pl.BlockSpec((NROW, NE), lambda i: (i, 0)),
            pl.BlockSpec((NROW, F), lambda i: (i, 0)),
            pl.BlockSpec((NROW, F), lambda i: (i, 0)),
            pl.BlockSpec((NROW, F), lambda i: (i, 0)),
            pl.BlockSpec((NE, F), lambda i: (0, 0)),
            pl.BlockSpec((1, F), lambda i: (0, 0)),
            pl.BlockSpec((F, 10), lambda i: (0, 0)),
            pl.BlockSpec((1, 10), lambda i: (0, 0)),
        ],
        out_specs=[
            pl.BlockSpec((NROW, F), lambda i: (i, 0)),
            pl.BlockSpec((NROW, F), lambda i: (i, 0)),
            pl.BlockSpec((NROW, 10), lambda i: (i, 0)),
        ],
        out_shape=[
            jax.ShapeDtypeStruct((N_REAL, F), _f32),
            jax.ShapeDtypeStruct((N_REAL, F), _f32),
            jax.ShapeDtypeStruct((N_REAL, 10), _f32),
        ],
    )
    d_hat, d_hat_adj, y_hat = head(
        node3n[:N_REAL], x, x_complete, mask.astype(jnp.int32),
        cmat, p["reph_b"].reshape(1, F), p["nph_W"],
        p["nph_b"].reshape(1, 10))

    return d_hat, d_hat_adj, y_hat, kl.reshape(()), probs
